# baseline (speedup 1.0000x reference)
import functools

import jax
import jax.numpy as jnp
from jax import lax
from jax.experimental import pallas as pl
from jax.experimental.pallas import tpu as pltpu
from jax.experimental.pallas import tpu_sc as plsc

_VOCAB = 100000
_C = 50
_CP = 64
_TW = 128
_B = 1024
_L = 200
_NW = 32
_BW = _B // _NW
_L0 = 128
_L1 = _L - _L0
_VB = 2048
_NVB = 49
_VP = _NVB * _VB


def _tr_body(w_ref, wt_ref):
  x = w_ref[...]
  eye = (jax.lax.broadcasted_iota(jnp.int32, (_C, _CP), 0)
         == jax.lax.broadcasted_iota(jnp.int32, (_C, _CP), 1)
         ).astype(jnp.float32)
  dims = (((0,), (0,)), ((), ()))
  wt_ref[:, : _CP] = jax.lax.dot_general(
      x[:, : _VB // 2], eye, dims, preferred_element_type=jnp.float32)
  wt_ref[:, _CP:] = jax.lax.dot_general(
      x[:, _VB // 2 :], eye, dims, preferred_element_type=jnp.float32)


def _make_table(W):
  return pl.pallas_call(
      _tr_body,
      grid=(_NVB,),
      in_specs=[pl.BlockSpec((_C, _VB), lambda i: (0, i))],
      out_specs=pl.BlockSpec((_VB // 2, _TW), lambda i: (i, 0)),
      out_shape=jax.ShapeDtypeStruct((_VP // 2, _TW), jnp.float32),
  )(W)


def _sc_body(wt_hbm, ids_hbm, len_hbm, bias_hbm, out_hbm,
             idx_v, buf0_v, buf1_v, out_v, len_v, inv_v, bias_v, sem0, sem1):
  wid = lax.axis_index("s") * 2 + lax.axis_index("c")
  base = wid * _BW

  pltpu.sync_copy(ids_hbm.at[pl.ds(base, _BW)], idx_v)
  pltpu.sync_copy(len_hbm.at[pl.ds(base, _BW)], len_v)
  pltpu.sync_copy(bias_hbm, bias_v)

  for c in range(_BW // 16):
    lens = len_v[pl.ds(c * 16, 16)].astype(jnp.float32)
    inv_v[pl.ds(c * 16, 16)] = 1.0 / lens

  bias = [bias_v[pl.ds(c * 16, 16)] for c in range(_CP // 16)]
  bufs = (buf0_v, buf1_v)
  sems = (sem0, sem1)

  def issue(r, buf, sem):
    pltpu.async_copy(wt_hbm.at[idx_v.at[r, pl.ds(0, _L0)]],
                     buf.at[pl.ds(0, _L0)], sem)
    pltpu.async_copy(wt_hbm.at[idx_v.at[r, pl.ds(_L0, _L1)]],
                     buf.at[pl.ds(_L0, _L1)], sem)

  def drain(r, buf, sem):
    pltpu.make_async_copy(wt_hbm.at[idx_v.at[r, pl.ds(0, _L0)]],
                          buf.at[pl.ds(0, _L0)], sem).wait()
    pltpu.make_async_copy(wt_hbm.at[idx_v.at[r, pl.ds(_L0, _L1)]],
                          buf.at[pl.ds(_L0, _L1)], sem).wait()

  def consume(r, buf):
    def acc_step(j8, accs):
      out = list(accs)
      for k in range(8):
        j = j8 * 8 + k
        for c in range(_CP // 16):
          out[c] = out[c] + buf[j, pl.ds(c * 16, 16)]
      return tuple(out)

    zeros = tuple(jnp.zeros((16,), jnp.float32) for _ in range(_CP // 16))
    accs = lax.fori_loop(0, _L // 8, acc_step, zeros, unroll=False)

    inv = jnp.full((16,), inv_v[pl.ds(r, 16)][0], jnp.float32)
    for c in range(_CP // 16):
      out_v[r, pl.ds(c * 16, 16)] = accs[c] * inv + bias[c]

  issue(0, buf0_v, sem0)

  def row_pair(rp, carry):
    for k in range(2):
      r = 2 * rp + k

      @pl.when(r < _BW - 1)
      def _prefetch():
        issue(r + 1, bufs[1 - k], sems[1 - k])

      drain(r, bufs[k], sems[k])
      consume(r, bufs[k])
    return carry

  lax.fori_loop(0, _BW // 2, row_pair, 0, unroll=False)

  pltpu.sync_copy(out_v, out_hbm.at[pl.ds(base, _BW)])


@jax.jit
def _bow_logits(wt, ids, lens, bias):
  mesh = plsc.VectorSubcoreMesh(core_axis_name="c", subcore_axis_name="s")
  f = pl.kernel(
      _sc_body,
      out_type=jax.ShapeDtypeStruct((_B, _CP), jnp.float32),
      mesh=mesh,
      compiler_params=pltpu.CompilerParams(use_tc_tiling_on_sc=False),
      scratch_types=[
          pltpu.VMEM((_BW, _L), jnp.int32),
          pltpu.VMEM((_L, _CP), jnp.float32),
          pltpu.VMEM((_L, _CP), jnp.float32),
          pltpu.VMEM((_BW, _CP), jnp.float32),
          pltpu.VMEM((_BW,), jnp.int32),
          pltpu.VMEM((_BW + 16,), jnp.float32),
          pltpu.VMEM((_CP,), jnp.float32),
          pltpu.SemaphoreType.DMA,
          pltpu.SemaphoreType.DMA,
      ],
  )
  return f(wt, ids, lens, bias)


def kernel(seq_lengths, input_ids, W, b):
  wt = _make_table(W).reshape(_VP, _CP)
  bias = jnp.zeros((_CP,), jnp.float32).at[:_C].set(b)
  ids = input_ids.astype(jnp.int32)
  ids = (ids & ~2047) + 2 * (ids & 1023) + ((ids >> 10) & 1)
  out = _bow_logits(wt, ids, seq_lengths.astype(jnp.int32), bias)
  return out[:, :_C]


# speedup vs baseline: 1.1457x; 1.1457x over previous
import functools

import jax
import jax.numpy as jnp
from jax import lax
from jax.experimental import pallas as pl
from jax.experimental.pallas import tpu as pltpu
from jax.experimental.pallas import tpu_sc as plsc

_VOCAB = 100000
_C = 50
_CP = 64
_TW = 128
_B = 1024
_L = 200
_NW = 32
_BW = _B // _NW
_L0 = 128
_L1 = _L - _L0
_VB = 4096
_NVB = 25
_VP = _NVB * _VB


def _tr_body(w_ref, wt_ref):
  x = w_ref[...]
  xp = jnp.concatenate([x, jnp.zeros((_CP - _C, _VB), jnp.float32)], axis=0)
  wt_ref[:, : _CP] = xp[:, : _VB // 2].T
  wt_ref[:, _CP:] = xp[:, _VB // 2 :].T


def _make_table(W):
  return pl.pallas_call(
      _tr_body,
      grid=(_NVB,),
      in_specs=[pl.BlockSpec((_C, _VB), lambda i: (0, i))],
      out_specs=pl.BlockSpec((_VB // 2, _TW), lambda i: (i, 0)),
      out_shape=jax.ShapeDtypeStruct((_VP // 2, _TW), jnp.float32),
  )(W)


def _sc_body(wt_hbm, ids_hbm, len_hbm, bias_hbm, out_hbm,
             idx_v, buf0_v, buf1_v, out_v, len_v, inv_v, bias_v, sem0, sem1):
  wid = lax.axis_index("s") * 2 + lax.axis_index("c")
  base = wid * _BW

  pltpu.sync_copy(ids_hbm.at[pl.ds(base, _BW)], idx_v)
  pltpu.sync_copy(len_hbm.at[pl.ds(base, _BW)], len_v)
  pltpu.sync_copy(bias_hbm, bias_v)

  for c in range(_BW // 16):
    lens = len_v[pl.ds(c * 16, 16)].astype(jnp.float32)
    inv_v[pl.ds(c * 16, 16)] = 1.0 / lens

  bias = [bias_v[pl.ds(c * 16, 16)] for c in range(_CP // 16)]
  bufs = (buf0_v, buf1_v)
  sems = (sem0, sem1)

  def issue(r, buf, sem):
    pltpu.async_copy(wt_hbm.at[idx_v.at[r, pl.ds(0, _L0)]],
                     buf.at[pl.ds(0, _L0)], sem)
    pltpu.async_copy(wt_hbm.at[idx_v.at[r, pl.ds(_L0, _L1)]],
                     buf.at[pl.ds(_L0, _L1)], sem)

  def drain(r, buf, sem):
    pltpu.make_async_copy(wt_hbm.at[idx_v.at[r, pl.ds(0, _L0)]],
                          buf.at[pl.ds(0, _L0)], sem).wait()
    pltpu.make_async_copy(wt_hbm.at[idx_v.at[r, pl.ds(_L0, _L1)]],
                          buf.at[pl.ds(_L0, _L1)], sem).wait()

  def consume(r, buf):
    def acc_step(j8, accs):
      out = list(accs)
      for k in range(8):
        j = j8 * 8 + k
        for c in range(_CP // 16):
          out[c] = out[c] + buf[j, pl.ds(c * 16, 16)]
      return tuple(out)

    zeros = tuple(jnp.zeros((16,), jnp.float32) for _ in range(_CP // 16))
    accs = lax.fori_loop(0, _L // 8, acc_step, zeros, unroll=False)

    inv = jnp.full((16,), inv_v[pl.ds(r, 16)][0], jnp.float32)
    for c in range(_CP // 16):
      out_v[r, pl.ds(c * 16, 16)] = accs[c] * inv + bias[c]

  issue(0, buf0_v, sem0)

  def row_pair(rp, carry):
    for k in range(2):
      r = 2 * rp + k

      @pl.when(r < _BW - 1)
      def _prefetch():
        issue(r + 1, bufs[1 - k], sems[1 - k])

      drain(r, bufs[k], sems[k])
      consume(r, bufs[k])
    return carry

  lax.fori_loop(0, _BW // 2, row_pair, 0, unroll=False)

  pltpu.sync_copy(out_v, out_hbm.at[pl.ds(base, _BW)])


@jax.jit
def _bow_logits(wt, ids, lens, bias):
  mesh = plsc.VectorSubcoreMesh(core_axis_name="c", subcore_axis_name="s")
  f = pl.kernel(
      _sc_body,
      out_type=jax.ShapeDtypeStruct((_B, _CP), jnp.float32),
      mesh=mesh,
      compiler_params=pltpu.CompilerParams(use_tc_tiling_on_sc=False),
      scratch_types=[
          pltpu.VMEM((_BW, _L), jnp.int32),
          pltpu.VMEM((_L, _CP), jnp.float32),
          pltpu.VMEM((_L, _CP), jnp.float32),
          pltpu.VMEM((_BW, _CP), jnp.float32),
          pltpu.VMEM((_BW,), jnp.int32),
          pltpu.VMEM((_BW + 16,), jnp.float32),
          pltpu.VMEM((_CP,), jnp.float32),
          pltpu.SemaphoreType.DMA,
          pltpu.SemaphoreType.DMA,
      ],
  )
  return f(wt, ids, lens, bias)


def kernel(seq_lengths, input_ids, W, b):
  wt = _make_table(W).reshape(_VP, _CP)
  bias = jnp.zeros((_CP,), jnp.float32).at[:_C].set(b)
  ids = input_ids.astype(jnp.int32)
  hb = _VB // 2
  ids = (ids & ~(_VB - 1)) + 2 * (ids & (hb - 1)) + ((ids // hb) & 1)
  out = _bow_logits(wt, ids, seq_lengths.astype(jnp.int32), bias)
  return out[:, :_C]


# speedup vs baseline: 1.2161x; 1.0615x over previous
import functools

import jax
import jax.numpy as jnp
from jax import lax
from jax.experimental import pallas as pl
from jax.experimental.pallas import tpu as pltpu
from jax.experimental.pallas import tpu_sc as plsc

_VOCAB = 100000
_C = 50
_CP = 64
_TW = 128
_B = 1024
_L = 200
_NW = 32
_BW = _B // _NW
_L0 = 128
_L1 = _L - _L0
_VB = 8192
_NVB = 13
_VP = _NVB * _VB


def _tr_body(w_ref, wt_ref):
  x = w_ref[...]
  xp = jnp.concatenate([x, jnp.zeros((_CP - _C, _VB), jnp.float32)], axis=0)
  wt_ref[:, : _CP] = xp[:, : _VB // 2].T
  wt_ref[:, _CP:] = xp[:, _VB // 2 :].T


def _make_table(W):
  return pl.pallas_call(
      _tr_body,
      grid=(_NVB,),
      in_specs=[pl.BlockSpec((_C, _VB), lambda i: (0, i))],
      out_specs=pl.BlockSpec((_VB // 2, _TW), lambda i: (i, 0)),
      out_shape=jax.ShapeDtypeStruct((_VP // 2, _TW), jnp.float32),
  )(W)


def _sc_body(wt_hbm, ids_hbm, len_hbm, bias_hbm, out_hbm,
             idx_v, buf0_v, buf1_v, out_v, len_v, inv_v, bias_v, sem0, sem1):
  wid = lax.axis_index("s") * 2 + lax.axis_index("c")
  base = wid * _BW

  pltpu.sync_copy(ids_hbm.at[pl.ds(base, _BW)], idx_v)
  pltpu.sync_copy(len_hbm.at[pl.ds(base, _BW)], len_v)
  pltpu.sync_copy(bias_hbm, bias_v)

  for c in range(_BW // 16):
    lens = len_v[pl.ds(c * 16, 16)].astype(jnp.float32)
    inv_v[pl.ds(c * 16, 16)] = 1.0 / lens

  bias = [bias_v[pl.ds(c * 16, 16)] for c in range(_CP // 16)]
  bufs = (buf0_v, buf1_v)
  sems = (sem0, sem1)

  def issue(r, buf, sem):
    pltpu.async_copy(wt_hbm.at[idx_v.at[r, pl.ds(0, _L0)]],
                     buf.at[pl.ds(0, _L0)], sem)
    pltpu.async_copy(wt_hbm.at[idx_v.at[r, pl.ds(_L0, _L1)]],
                     buf.at[pl.ds(_L0, _L1)], sem)

  def drain(r, buf, sem):
    pltpu.make_async_copy(wt_hbm.at[idx_v.at[r, pl.ds(0, _L0)]],
                          buf.at[pl.ds(0, _L0)], sem).wait()
    pltpu.make_async_copy(wt_hbm.at[idx_v.at[r, pl.ds(_L0, _L1)]],
                          buf.at[pl.ds(_L0, _L1)], sem).wait()

  def consume(r, buf):
    def acc_step(j8, accs):
      out = list(accs)
      for k in range(8):
        j = j8 * 8 + k
        for c in range(_CP // 16):
          out[c] = out[c] + buf[j, pl.ds(c * 16, 16)]
      return tuple(out)

    zeros = tuple(jnp.zeros((16,), jnp.float32) for _ in range(_CP // 16))
    accs = lax.fori_loop(0, _L // 8, acc_step, zeros, unroll=False)

    inv = jnp.full((16,), inv_v[pl.ds(r, 16)][0], jnp.float32)
    for c in range(_CP // 16):
      out_v[r, pl.ds(c * 16, 16)] = accs[c] * inv + bias[c]

  issue(0, buf0_v, sem0)

  def row_pair(rp, carry):
    for k in range(2):
      r = 2 * rp + k

      @pl.when(r < _BW - 1)
      def _prefetch():
        issue(r + 1, bufs[1 - k], sems[1 - k])

      drain(r, bufs[k], sems[k])
      consume(r, bufs[k])
    return carry

  lax.fori_loop(0, _BW // 2, row_pair, 0, unroll=False)

  pltpu.sync_copy(out_v, out_hbm.at[pl.ds(base, _BW)])


@jax.jit
def _bow_logits(wt, ids, lens, bias):
  mesh = plsc.VectorSubcoreMesh(core_axis_name="c", subcore_axis_name="s")
  f = pl.kernel(
      _sc_body,
      out_type=jax.ShapeDtypeStruct((_B, _CP), jnp.float32),
      mesh=mesh,
      compiler_params=pltpu.CompilerParams(use_tc_tiling_on_sc=False),
      scratch_types=[
          pltpu.VMEM((_BW, _L), jnp.int32),
          pltpu.VMEM((_L, _CP), jnp.float32),
          pltpu.VMEM((_L, _CP), jnp.float32),
          pltpu.VMEM((_BW, _CP), jnp.float32),
          pltpu.VMEM((_BW,), jnp.int32),
          pltpu.VMEM((_BW + 16,), jnp.float32),
          pltpu.VMEM((_CP,), jnp.float32),
          pltpu.SemaphoreType.DMA,
          pltpu.SemaphoreType.DMA,
      ],
  )
  return f(wt, ids, lens, bias)


def kernel(seq_lengths, input_ids, W, b):
  wt = _make_table(W).reshape(_VP, _CP)
  bias = jnp.zeros((_CP,), jnp.float32).at[:_C].set(b)
  ids = input_ids.astype(jnp.int32)
  hb = _VB // 2
  ids = (ids & ~(_VB - 1)) + 2 * (ids & (hb - 1)) + ((ids // hb) & 1)
  out = _bow_logits(wt, ids, seq_lengths.astype(jnp.int32), bias)
  return out[:, :_C]


# speedup vs baseline: 1.2182x; 1.0017x over previous
import functools

import jax
import jax.numpy as jnp
from jax import lax
from jax.experimental import pallas as pl
from jax.experimental.pallas import tpu as pltpu
from jax.experimental.pallas import tpu_sc as plsc

_VOCAB = 100000
_C = 50
_CP = 64
_TW = 128
_B = 1024
_L = 200
_NW = 32
_BW = _B // _NW
_L0 = 128
_L1 = _L - _L0
_VB = 16384
_NVB = 7
_VP = _NVB * _VB


def _tr_body(w_ref, wt_ref):
  x = w_ref[...]
  xp = jnp.concatenate([x, jnp.zeros((_CP - _C, _VB), jnp.float32)], axis=0)
  wt_ref[:, : _CP] = xp[:, : _VB // 2].T
  wt_ref[:, _CP:] = xp[:, _VB // 2 :].T


def _make_table(W):
  return pl.pallas_call(
      _tr_body,
      grid=(_NVB,),
      in_specs=[pl.BlockSpec((_C, _VB), lambda i: (0, i))],
      out_specs=pl.BlockSpec((_VB // 2, _TW), lambda i: (i, 0)),
      out_shape=jax.ShapeDtypeStruct((_VP // 2, _TW), jnp.float32),
  )(W)


def _sc_body(wt_hbm, ids_hbm, len_hbm, bias_hbm, out_hbm,
             idx_v, buf0_v, buf1_v, out_v, len_v, inv_v, bias_v, sem0, sem1):
  wid = lax.axis_index("s") * 2 + lax.axis_index("c")
  base = wid * _BW

  pltpu.sync_copy(ids_hbm.at[pl.ds(base, _BW)], idx_v)
  pltpu.sync_copy(len_hbm.at[pl.ds(base, _BW)], len_v)
  pltpu.sync_copy(bias_hbm, bias_v)

  for c in range(_BW // 16):
    lens = len_v[pl.ds(c * 16, 16)].astype(jnp.float32)
    inv_v[pl.ds(c * 16, 16)] = 1.0 / lens

  bias = [bias_v[pl.ds(c * 16, 16)] for c in range(_CP // 16)]
  bufs = (buf0_v, buf1_v)
  sems = (sem0, sem1)

  def issue(r, buf, sem):
    pltpu.async_copy(wt_hbm.at[idx_v.at[r, pl.ds(0, _L0)]],
                     buf.at[pl.ds(0, _L0)], sem)
    pltpu.async_copy(wt_hbm.at[idx_v.at[r, pl.ds(_L0, _L1)]],
                     buf.at[pl.ds(_L0, _L1)], sem)

  def drain(r, buf, sem):
    pltpu.make_async_copy(wt_hbm.at[idx_v.at[r, pl.ds(0, _L0)]],
                          buf.at[pl.ds(0, _L0)], sem).wait()
    pltpu.make_async_copy(wt_hbm.at[idx_v.at[r, pl.ds(_L0, _L1)]],
                          buf.at[pl.ds(_L0, _L1)], sem).wait()

  def consume(r, buf):
    def acc_step(j8, accs):
      out = list(accs)
      for k in range(8):
        j = j8 * 8 + k
        for c in range(_CP // 16):
          out[c] = out[c] + buf[j, pl.ds(c * 16, 16)]
      return tuple(out)

    zeros = tuple(jnp.zeros((16,), jnp.float32) for _ in range(_CP // 16))
    accs = lax.fori_loop(0, _L // 8, acc_step, zeros, unroll=False)

    inv = jnp.full((16,), inv_v[pl.ds(r, 16)][0], jnp.float32)
    for c in range(_CP // 16):
      out_v[r, pl.ds(c * 16, 16)] = accs[c] * inv + bias[c]

  issue(0, buf0_v, sem0)

  def row_pair(rp, carry):
    for k in range(2):
      r = 2 * rp + k

      @pl.when(r < _BW - 1)
      def _prefetch():
        issue(r + 1, bufs[1 - k], sems[1 - k])

      drain(r, bufs[k], sems[k])
      consume(r, bufs[k])
    return carry

  lax.fori_loop(0, _BW // 2, row_pair, 0, unroll=False)

  pltpu.sync_copy(out_v, out_hbm.at[pl.ds(base, _BW)])


@jax.jit
def _bow_logits(wt, ids, lens, bias):
  mesh = plsc.VectorSubcoreMesh(core_axis_name="c", subcore_axis_name="s")
  f = pl.kernel(
      _sc_body,
      out_type=jax.ShapeDtypeStruct((_B, _CP), jnp.float32),
      mesh=mesh,
      compiler_params=pltpu.CompilerParams(use_tc_tiling_on_sc=False),
      scratch_types=[
          pltpu.VMEM((_BW, _L), jnp.int32),
          pltpu.VMEM((_L, _CP), jnp.float32),
          pltpu.VMEM((_L, _CP), jnp.float32),
          pltpu.VMEM((_BW, _CP), jnp.float32),
          pltpu.VMEM((_BW,), jnp.int32),
          pltpu.VMEM((_BW + 16,), jnp.float32),
          pltpu.VMEM((_CP,), jnp.float32),
          pltpu.SemaphoreType.DMA,
          pltpu.SemaphoreType.DMA,
      ],
  )
  return f(wt, ids, lens, bias)


def kernel(seq_lengths, input_ids, W, b):
  wt = _make_table(W).reshape(_VP, _CP)
  bias = jnp.zeros((_CP,), jnp.float32).at[:_C].set(b)
  ids = input_ids.astype(jnp.int32)
  hb = _VB // 2
  ids = (ids & ~(_VB - 1)) + 2 * (ids & (hb - 1)) + ((ids // hb) & 1)
  out = _bow_logits(wt, ids, seq_lengths.astype(jnp.int32), bias)
  return out[:, :_C]
